# baseline (speedup 1.0000x reference)
import functools

import jax
import jax.numpy as jnp
from jax import lax
from jax.experimental import pallas as pl
from jax.experimental.pallas import tpu as pltpu
from jax.experimental.pallas import tpu_sc as plsc

_NUM_CORES = 2
_NUM_SUBCORES = 16
_NUM_WORKERS = _NUM_CORES * _NUM_SUBCORES


def _gather_body(rows_per_worker,
                 idx0_hbm, idx1_hbm, table_hbm,
                 out0_hbm, out1_hbm,
                 idx0_v, idx1_v, vals0_v, vals1_v, sem0, sem1):
    wid = lax.axis_index("s") * _NUM_CORES + lax.axis_index("c")
    base = wid * rows_per_worker
    sl = pl.ds(base, rows_per_worker)
    half = rows_per_worker // 2
    sla = pl.ds(base, half)
    slb = pl.ds(base + half, half)
    lo = pl.ds(0, half)
    hi = pl.ds(half, half)
    s0a = pltpu.async_copy(idx0_hbm.at[sla], idx0_v.at[lo], sem0)
    s1a = pltpu.async_copy(idx1_hbm.at[sla], idx1_v.at[lo], sem1)
    s0b = pltpu.async_copy(idx0_hbm.at[slb], idx0_v.at[hi], sem0)
    s1b = pltpu.async_copy(idx1_hbm.at[slb], idx1_v.at[hi], sem1)
    s0a.wait()
    g0a = pltpu.async_copy(table_hbm.at[idx0_v.at[lo]], vals0_v.at[lo], sem0)
    s1a.wait()
    g1a = pltpu.async_copy(table_hbm.at[idx1_v.at[lo]], vals1_v.at[lo], sem1)
    s0b.wait()
    g0b = pltpu.async_copy(table_hbm.at[idx0_v.at[hi]], vals0_v.at[hi], sem0)
    s1b.wait()
    g1b = pltpu.async_copy(table_hbm.at[idx1_v.at[hi]], vals1_v.at[hi], sem1)
    g0a.wait()
    g0b.wait()
    w0 = pltpu.async_copy(vals0_v, out0_hbm.at[sl], sem0)
    g1a.wait()
    g1b.wait()
    w1 = pltpu.async_copy(vals1_v, out1_hbm.at[sl], sem1)
    w0.wait()
    w1.wait()


def kernel(item_id, batch_size, item_bias):
    b, n = item_id.shape
    rows_per_worker = b // _NUM_WORKERS
    table = item_bias[:, 0]
    idx0 = item_id[:, 0]
    idx1 = item_id[:, 1]

    mesh = plsc.VectorSubcoreMesh(core_axis_name="c", subcore_axis_name="s")
    out0, out1 = pl.kernel(
        functools.partial(_gather_body, rows_per_worker),
        out_type=(
            jax.ShapeDtypeStruct((b,), jnp.float32),
            jax.ShapeDtypeStruct((b,), jnp.float32),
        ),
        mesh=mesh,
        scratch_types=[
            pltpu.VMEM((rows_per_worker,), jnp.int32),
            pltpu.VMEM((rows_per_worker,), jnp.int32),
            pltpu.VMEM((rows_per_worker,), jnp.float32),
            pltpu.VMEM((rows_per_worker,), jnp.float32),
            pltpu.SemaphoreType.DMA,
            pltpu.SemaphoreType.DMA,
        ],
    )(idx0, idx1, table)
    return jnp.stack([out0, out1], axis=-1)


# speedup vs baseline: 1.0063x; 1.0063x over previous
import functools

import jax
import jax.numpy as jnp
from jax import lax
from jax.experimental import pallas as pl
from jax.experimental.pallas import tpu as pltpu
from jax.experimental.pallas import tpu_sc as plsc

_NUM_CORES = 2
_NUM_SUBCORES = 16
_NUM_WORKERS = _NUM_CORES * _NUM_SUBCORES


def _gather_body(rows_per_worker,
                 idx0_hbm, idx1_hbm, table_hbm,
                 out0_hbm, out1_hbm,
                 idx0_v, idx1_v, vals0_v, vals1_v, sem0, sem1):
    wid = lax.axis_index("s") * _NUM_CORES + lax.axis_index("c")
    base = wid * rows_per_worker
    sl = pl.ds(base, rows_per_worker)
    s0 = pltpu.async_copy(idx0_hbm.at[sl], idx0_v, sem0)
    s1 = pltpu.async_copy(idx1_hbm.at[sl], idx1_v, sem1)
    s0.wait()
    g0 = pltpu.async_copy(table_hbm.at[idx0_v], vals0_v, sem0)
    s1.wait()
    g1 = pltpu.async_copy(table_hbm.at[idx1_v], vals1_v, sem1)
    g0.wait()
    w0 = pltpu.async_copy(vals0_v, out0_hbm.at[sl], sem0)
    g1.wait()
    w1 = pltpu.async_copy(vals1_v, out1_hbm.at[sl], sem1)
    w0.wait()
    w1.wait()


def kernel(item_id, batch_size, item_bias):
    b, n = item_id.shape
    rows_per_worker = b // _NUM_WORKERS
    table = item_bias[:, 0]
    idx0 = item_id[:, 0]
    idx1 = item_id[:, 1]

    mesh = plsc.VectorSubcoreMesh(core_axis_name="c", subcore_axis_name="s")
    out0, out1 = pl.kernel(
        functools.partial(_gather_body, rows_per_worker),
        out_type=(
            jax.ShapeDtypeStruct((b,), jnp.float32),
            jax.ShapeDtypeStruct((b,), jnp.float32),
        ),
        mesh=mesh,
        scratch_types=[
            pltpu.VMEM((rows_per_worker,), jnp.int32),
            pltpu.VMEM((rows_per_worker,), jnp.int32),
            pltpu.VMEM((rows_per_worker,), jnp.float32),
            pltpu.VMEM((rows_per_worker,), jnp.float32),
            pltpu.SemaphoreType.DMA,
            pltpu.SemaphoreType.DMA,
        ],
    )(idx0, idx1, table)
    return jnp.stack([out0, out1], axis=-1)
